# baseline (speedup 1.0000x reference)
import functools

import jax
import jax.numpy as jnp
from jax import lax
from jax.experimental import pallas as pl
from jax.experimental.pallas import tpu as pltpu
from jax.experimental.pallas import tpu_sc as plsc

_B, _N, _CODE_DIM, _K, _HIDDEN = 16, 1024, 256, 8192, 768
_COMMITMENT_COST = 0.25
_THRESHOLD = 0.5

_TOK = _B * _N


_info = plsc.get_sparse_core_info()
_NC, _NS = _info.num_cores, _info.num_subcores
_NW = _NC * _NS


def _make_sc_gather(tok_per_worker, chunk, row_dim):
    n_ch = tok_per_worker // chunk
    mesh = plsc.VectorSubcoreMesh(core_axis_name="c", subcore_axis_name="s")

    @functools.partial(
        pl.kernel,
        mesh=mesh,
        out_type=jax.ShapeDtypeStruct((_TOK, row_dim), jnp.float32),
        scratch_types=[
            pltpu.VMEM((n_ch, chunk), jnp.int32),
            pltpu.VMEM((chunk, row_dim), jnp.float32),
            pltpu.VMEM((chunk, row_dim), jnp.float32),
            pltpu.SemaphoreType.DMA,
            pltpu.SemaphoreType.DMA,
        ],
    )
    def sc_gather(table_hbm, idx_hbm, out_hbm, idx_v, rows0, rows1, sem0, sem1):
        wid = lax.axis_index("s") * _NC + lax.axis_index("c")
        base = wid * tok_per_worker
        pltpu.sync_copy(idx_hbm.at[wid], idx_v)
        bufs = (rows0, rows1)
        sems = (sem0, sem1)
        copies = [None, None]
        copies[0] = pltpu.async_copy(table_hbm.at[idx_v.at[0]], rows0, sem0)
        for c in range(n_ch):
            cur = c % 2
            if c + 1 < n_ch:
                nxt = (c + 1) % 2
                copies[nxt] = pltpu.async_copy(
                    table_hbm.at[idx_v.at[c + 1]], bufs[nxt], sems[nxt])
            copies[cur].wait()
            pltpu.sync_copy(bufs[cur], out_hbm.at[pl.ds(base + c * chunk, chunk)])

    return sc_gather


_PER_W = _TOK // _NW
_gather_code = _make_sc_gather(_PER_W, 128, _CODE_DIM)
_gather_proj = _make_sc_gather(_PER_W, 64, _HIDDEN)


_KBLK = 512


def _proj_body(c_ref, w_ref, b_ref, o_ref):
    o_ref[...] = lax.dot_general(
        c_ref[...], w_ref[...], (((1,), (1,)), ((), ())),
        preferred_element_type=jnp.float32) + b_ref[...]


_proj_call = pl.pallas_call(
    _proj_body,
    grid=(_K // _KBLK,),
    in_specs=[
        pl.BlockSpec((_KBLK, _CODE_DIM), lambda k: (k, 0)),
        pl.BlockSpec((_HIDDEN, _CODE_DIM), lambda k: (0, 0)),
        pl.BlockSpec((1, _HIDDEN), lambda k: (0, 0)),
    ],
    out_specs=pl.BlockSpec((_KBLK, _HIDDEN), lambda k: (k, 0)),
    out_shape=jax.ShapeDtypeStruct((_K, _HIDDEN), jnp.float32),
)


_SUB = _N // 128


def _stats_body(q_ref, m_ref, valid_ref, loss_ref, acc_ref):
    bidx = pl.program_id(0)
    q3 = q_ref[0].reshape(_SUB, 128, _CODE_DIM)
    m3 = m_ref[0].reshape(_SUB, 128, _CODE_DIM)
    sim = jnp.sum(q3 * m3, axis=2)
    sq = jnp.sum((m3 - q3) ** 2, axis=2)

    neg = jnp.float32(-jnp.inf)
    cur = jnp.float32(jnp.inf)
    removed = jnp.float32(0.0)
    for _ in range(5):
        mmax = jnp.max(jnp.where(sim < cur, sim, neg))
        cnt_eq = jnp.sum(jnp.where(sim == mmax, 1.0, 0.0))
        upd = removed < 5.0
        removed = jnp.where(upd, removed + cnt_eq, removed)
        cur = jnp.where(upd, mmax, cur)

    thresh = jnp.minimum(cur, jnp.float32(_THRESHOLD))
    validf = (sim >= thresh).astype(jnp.float32)
    valid_ref[0] = validf.astype(jnp.int32)

    num = jnp.sum(sq * validf)
    cnt = jnp.sum(validf)

    @pl.when(bidx == 0)
    def _init():
        acc_ref[0] = num
        acc_ref[1] = cnt

    @pl.when(bidx > 0)
    def _accum():
        acc_ref[0] = acc_ref[0] + num
        acc_ref[1] = acc_ref[1] + cnt

    @pl.when(bidx == _B - 1)
    def _final():
        denom = acc_ref[1] * jnp.float32(_CODE_DIM)
        loss = (1.0 + _COMMITMENT_COST) * acc_ref[0] / denom
        loss_ref[...] = jnp.full((1, 1), loss, jnp.float32)


_stats_call = pl.pallas_call(
    _stats_body,
    grid=(_B,),
    in_specs=[
        pl.BlockSpec((1, _N, _CODE_DIM), lambda b: (b, 0, 0)),
        pl.BlockSpec((1, _N, _CODE_DIM), lambda b: (b, 0, 0)),
    ],
    out_specs=[
        pl.BlockSpec((1, _SUB, 128), lambda b: (b, 0, 0)),
        pl.BlockSpec((1, 1), lambda b: (0, 0)),
    ],
    out_shape=[
        jax.ShapeDtypeStruct((_B, _SUB, 128), jnp.int32),
        jax.ShapeDtypeStruct((1, 1), jnp.float32),
    ],
    scratch_shapes=[pltpu.SMEM((2,), jnp.float32)],
)


def kernel(mlc_proj, code, code_id, W, b):
    idx_code = code_id.reshape(_NW, _PER_W // 128, 128).astype(jnp.int32)
    idx_proj = code_id.reshape(_NW, _PER_W // 64, 64).astype(jnp.int32)

    code_pr = _proj_call(code, W, b.reshape(1, _HIDDEN))
    quant_flat = _gather_code(code, idx_code)
    out_flat = _gather_proj(code_pr, idx_proj)
    quant = quant_flat.reshape(_B, _N, _CODE_DIM)
    valid3, loss = _stats_call(quant, mlc_proj)
    valid = valid3.reshape(_B, _N) != 0
    return out_flat.reshape(_B, _N, _HIDDEN), valid, loss.reshape(())
